# Initial kernel scaffold; baseline (speedup 1.0000x reference)
import jax
import jax.numpy as jnp
from jax.experimental import pallas as pl


def kernel(positions, rank_embed, file_embed, flag_embed):
    raise NotImplementedError("write your pallas kernel here")


# speedup vs baseline: 2.6008x; 2.6008x over previous
import functools

import jax
import jax.numpy as jnp
from jax import lax
from jax.experimental import pallas as pl
from jax.experimental.pallas import tpu as pltpu
from jax.experimental.pallas import tpu_sc as plsc

D_MODEL = 2048
S = 8192
TABLE_ROWS = 72

NC = 2
NS = 16
NW = NC * NS
B_PER_W = S // NW
CH = 16
NCH = B_PER_W // CH


def _table_body(rank_ref, file_ref, flag_ref, out_ref):
    rank_part = jnp.concatenate(
        [jnp.broadcast_to(rank_ref[k:k + 1, :], (8, D_MODEL)) for k in range(8)],
        axis=0)
    file_part = jnp.concatenate([file_ref[...]] * 8, axis=0)
    out_ref[0:64, :] = rank_part + file_part
    out_ref[64:72, :] = flag_ref[...]


def _build_table(rank_embed, file_embed, flag_pad):
    return pl.pallas_call(
        _table_body,
        out_shape=jax.ShapeDtypeStruct((TABLE_ROWS, D_MODEL), jnp.float32),
    )(rank_embed, file_embed, flag_pad)


_mesh = plsc.VectorSubcoreMesh(core_axis_name="c", subcore_axis_name="s")


@functools.partial(
    pl.kernel,
    mesh=_mesh,
    out_type=jax.ShapeDtypeStruct((S, D_MODEL), jnp.float32),
    scratch_types=[
        pltpu.VMEM((NCH, CH), jnp.int32),
        pltpu.VMEM((CH, D_MODEL), jnp.float32),
        pltpu.VMEM((CH, D_MODEL), jnp.float32),
        pltpu.SemaphoreType.DMA,
        pltpu.SemaphoreType.DMA,
    ],
)
def _gather_kernel(idx_hbm, table_hbm, out_hbm, idx_v, buf0, buf1, sem0, sem1):
    wid = lax.axis_index("s") * NC + lax.axis_index("c")
    base = wid * B_PER_W
    pltpu.sync_copy(idx_hbm.at[wid], idx_v)
    bufs = (buf0, buf1)
    sems = (sem0, sem1)
    copies = [None, None]
    copies[0] = pltpu.async_copy(table_hbm.at[idx_v.at[0]], buf0, sem0)
    for c in range(NCH):
        s = c % 2
        if c + 1 < NCH:
            s2 = (c + 1) % 2
            copies[s2] = pltpu.async_copy(
                table_hbm.at[idx_v.at[c + 1]], bufs[s2], sems[s2])
        copies[s].wait()
        pltpu.sync_copy(bufs[s], out_hbm.at[pl.ds(base + c * CH, CH)])


def kernel(positions, rank_embed, file_embed, flag_embed):
    positions = positions.astype(jnp.int32)
    flag_pad = jnp.concatenate(
        [flag_embed.astype(jnp.float32), jnp.zeros((3, D_MODEL), jnp.float32)],
        axis=0)
    table = _build_table(rank_embed.astype(jnp.float32),
                         file_embed.astype(jnp.float32), flag_pad)
    idx = positions.reshape(NW, NCH, CH)
    return _gather_kernel(idx, table)
